# Initial kernel scaffold; baseline (speedup 1.0000x reference)
import jax
import jax.numpy as jnp
from jax.experimental import pallas as pl


def kernel(xyz, points, mlp_w0, mlp_b0, mlp_w1, mlp_b1, dn_w0, dn_b0, dn_w1, dn_b1, dn_w2, dn_b2, wn_w0, wn_b0, wn_w1, wn_b1, wn_w2, wn_b2, lin_w, lin_b):
    raise NotImplementedError("write your pallas kernel here")


# speedup vs baseline: 12.5001x; 12.5001x over previous
import functools

import jax
import jax.numpy as jnp
from jax import lax
from jax.experimental import pallas as pl
from jax.experimental.pallas import tpu as pltpu
from jax.experimental.pallas import tpu_sc as plsc

N = 4096
NPOINT = 1024
NSAMPLE = 32
BANDWIDTH = 0.1
EPS = 1e-5


def _density_body(xyz_rows_ref, xyz_cols_ref, dens_ref):
    q = xyz_rows_ref[...]
    c = xyz_cols_ref[...]
    mm = jax.lax.dot_general(q, c, (((1,), (0,)), ((), ())),
                             preferred_element_type=jnp.float32)
    qsq = (q[:, 0:1] * q[:, 0:1] + q[:, 1:2] * q[:, 1:2]
           + q[:, 2:3] * q[:, 2:3])
    csq = (c[0:1, :] * c[0:1, :] + c[1:2, :] * c[1:2, :]
           + c[2:3, :] * c[2:3, :])
    sq = (-2.0 * mm + qsq) + csq
    g = jnp.exp(sq * (-1.0 / (2.0 * BANDWIDTH * BANDWIDTH))) / (2.5 * BANDWIDTH)
    dens_ref[...] = jnp.mean(g, axis=1, keepdims=True)


def _density(xyz_rows, xyz):
    B = xyz.shape[0]
    RT = 512
    out = pl.pallas_call(
        _density_body,
        grid=(B, N // RT),
        in_specs=[
            pl.BlockSpec((None, RT, 3), lambda b, r: (b, r, 0)),
            pl.BlockSpec((None, 3, N), lambda b, r: (b, 0, 0)),
        ],
        out_specs=pl.BlockSpec((None, RT, 1), lambda b, r: (b, r, 0)),
        out_shape=jax.ShapeDtypeStruct((B, N, 1), jnp.float32),
    )(xyz_rows, xyz)
    return out.reshape(B, N)


def _densmlp_body(dens_ref, w0_ref, b0_ref, w1_ref, b1_ref, w2_ref, b2_ref,
                  out_ref):
    def inorm_relu(h):
        mu = jnp.mean(h, axis=1, keepdims=True)
        v = jnp.mean((h - mu) * (h - mu), axis=1, keepdims=True)
        return jnp.maximum((h - mu) / jnp.sqrt(v + EPS), 0.0)

    d = dens_ref[...]
    h = w0_ref[...] * d
    h = inorm_relu(h + b0_ref[...])
    h = jnp.dot(w1_ref[...], h, preferred_element_type=jnp.float32)
    h = inorm_relu(h + b1_ref[...])
    h = jnp.sum(w2_ref[...] * h, axis=0, keepdims=True)
    out_ref[...] = inorm_relu(h + b2_ref[...])


def _density_scale(dens, dn_w0, dn_b0, dn_w1, dn_b1, dn_w2, dn_b2):
    B = dens.shape[0]
    wspec = lambda shp: pl.BlockSpec(shp, lambda b: tuple(0 for _ in shp))
    return pl.pallas_call(
        _densmlp_body,
        grid=(B,),
        in_specs=[
            pl.BlockSpec((None, 1, N), lambda b: (b, 0, 0)),
            wspec((8, 1)), wspec((8, 1)),
            wspec((8, 8)), wspec((8, 1)),
            wspec((8, 1)), wspec((1, 1)),
        ],
        out_specs=pl.BlockSpec((None, 1, N), lambda b: (b, 0, 0)),
        out_shape=jax.ShapeDtypeStruct((B, 1, N), jnp.float32),
    )(dens.reshape(B, 1, N), dn_w0, dn_b0[:, None], dn_w1, dn_b1[:, None],
      jnp.transpose(dn_w2), dn_b2[:, None]).reshape(B, N)


def _fps_body(xyz_ref, nxc_ref, nxr_ref):
    x = xyz_ref[:, 0, :]
    y = xyz_ref[:, 1, :]
    z = xyz_ref[:, 2, :]
    B = x.shape[0]
    lane = jax.lax.broadcasted_iota(jnp.int32, (B, N), 1)
    piota = jax.lax.broadcasted_iota(jnp.int32, (1, NPOINT), 1)
    big = jnp.int32(N + 1)

    def body(i, state):
        distance, far, cxs, cys, czs = state
        m = (lane == far).astype(jnp.float32)
        cx = jnp.sum(x * m, axis=1, keepdims=True)
        cy = jnp.sum(y * m, axis=1, keepdims=True)
        cz = jnp.sum(z * m, axis=1, keepdims=True)
        oh = (piota == i).astype(jnp.float32)
        cxs = cxs + cx * oh
        cys = cys + cy * oh
        czs = czs + cz * oh
        dx = x - cx
        dy = y - cy
        dz = z - cz
        dist = dx * dx + dy * dy + dz * dz
        distance = jnp.minimum(distance, dist)
        mx = jnp.max(distance, axis=1, keepdims=True)
        far2 = jnp.min(jnp.where(distance == mx, lane, big), axis=1,
                       keepdims=True)
        return distance, far2, cxs, cys, czs

    zero = jnp.zeros((B, NPOINT), jnp.float32)
    init = (jnp.full((B, N), 1e10, jnp.float32),
            jnp.zeros((B, 1), jnp.int32), zero, zero, zero)
    _, _, cxs, cys, czs = jax.lax.fori_loop(0, NPOINT, body, init)
    nxc_ref[:, 0, :] = cxs
    nxc_ref[:, 1, :] = cys
    nxc_ref[:, 2, :] = czs
    nxr_ref[:, :, 0] = cxs
    nxr_ref[:, :, 1] = cys
    nxr_ref[:, :, 2] = czs


def _fps(xyz):
    B = xyz.shape[0]
    return pl.pallas_call(
        _fps_body,
        in_specs=[pl.BlockSpec(xyz.shape, lambda: (0, 0, 0))],
        out_specs=[
            pl.BlockSpec((B, 3, NPOINT), lambda: (0, 0, 0)),
            pl.BlockSpec((B, NPOINT, 3), lambda: (0, 0, 0)),
        ],
        out_shape=[
            jax.ShapeDtypeStruct((B, 3, NPOINT), jnp.float32),
            jax.ShapeDtypeStruct((B, NPOINT, 3), jnp.float32),
        ],
    )(xyz)


def _knn_body(q_ref, xyz_ref, idx_ref):
    q = q_ref[...]
    c = xyz_ref[...]
    mm = jax.lax.dot_general(q, c, (((1,), (0,)), ((), ())),
                             preferred_element_type=jnp.float32)
    qsq = (q[:, 0:1] * q[:, 0:1] + q[:, 1:2] * q[:, 1:2]
           + q[:, 2:3] * q[:, 2:3])
    csq = (c[0:1, :] * c[0:1, :] + c[1:2, :] * c[1:2, :]
           + c[2:3, :] * c[2:3, :])
    sq = (-2.0 * mm + qsq) + csq
    QT = q.shape[0]
    lane = jax.lax.broadcasted_iota(jnp.int32, (QT, N), 1)
    big = jnp.int32(N + 1)
    for s in range(NSAMPLE):
        mn = jnp.min(sq, axis=1, keepdims=True)
        idx = jnp.min(jnp.where(sq == mn, lane, big), axis=1, keepdims=True)
        idx_ref[:, s:s + 1] = idx
        sq = jnp.where(lane == idx, jnp.float32(3e38), sq)


def _knn(new_xyz_rows, xyz):
    B = xyz.shape[0]
    QT = 256
    return pl.pallas_call(
        _knn_body,
        grid=(B, NPOINT // QT),
        in_specs=[
            pl.BlockSpec((None, QT, 3), lambda b, t: (b, t, 0)),
            pl.BlockSpec((None, 3, N), lambda b, t: (b, 0, 0)),
        ],
        out_specs=pl.BlockSpec((None, QT, NSAMPLE), lambda b, t: (b, t, 0)),
        out_shape=jax.ShapeDtypeStruct((B, NPOINT, NSAMPLE), jnp.int32),
    )(new_xyz_rows, xyz)


def _sc_gather(table, gidx):
    TOT = gidx.shape[0]
    D = table.shape[1]
    info = plsc.get_sparse_core_info()
    NC, NS = info.num_cores, info.num_subcores
    NW = NC * NS
    per_w = TOT // NW
    CH = 512
    n_ch = per_w // CH
    mesh = plsc.VectorSubcoreMesh(core_axis_name="c", subcore_axis_name="s")

    @functools.partial(
        pl.kernel,
        mesh=mesh,
        out_type=jax.ShapeDtypeStruct((TOT, D), jnp.float32),
        scratch_types=[
            pltpu.VMEM((CH,), jnp.int32),
            pltpu.VMEM((CH, D), jnp.float32),
            pltpu.SemaphoreType.DMA,
        ],
    )
    def k(table_hbm, idx_hbm, out_hbm, idx_v, rows_v, sem):
        wid = lax.axis_index("s") * NC + lax.axis_index("c")
        base = wid * per_w
        for c in range(n_ch):
            off = base + c * CH
            pltpu.sync_copy(idx_hbm.at[pl.dslice(off, CH)], idx_v)
            pltpu.async_copy(table_hbm.at[idx_v], rows_v, sem).wait()
            pltpu.sync_copy(rows_v, out_hbm.at[pl.dslice(off, CH)])

    return k(table, gidx)


def _tr_body(g_ref, nx_ref, out_ref):
    g = g_ref[...]
    eye = (jax.lax.broadcasted_iota(jnp.int32, (128, 128), 0)
           == jax.lax.broadcasted_iota(jnp.int32, (128, 128), 1)
           ).astype(jnp.float32)
    gt = jax.lax.dot_general(eye, g, (((1,), (1,)), ((), ())),
                             preferred_element_type=jnp.float32)
    nx = nx_ref[...]
    nxrep = jnp.concatenate([nx] * (g.shape[0] // NPOINT), axis=1)
    xyzn = gt[0:3, :] - nxrep
    out_ref[...] = jnp.concatenate([xyzn, gt[3:4, :], gt[16:80, :]], axis=0)


def _transpose_prep(g, nx_cm):
    B = g.shape[0]
    R = NSAMPLE * NPOINT
    RT = 4 * NPOINT
    return pl.pallas_call(
        _tr_body,
        grid=(B, R // RT),
        in_specs=[
            pl.BlockSpec((None, RT, 128), lambda b, t: (b, t, 0)),
            pl.BlockSpec((None, 3, NPOINT), lambda b, t: (b, 0, 0)),
        ],
        out_specs=pl.BlockSpec((None, 68, RT), lambda b, t: (b, 0, t)),
        out_shape=jax.ShapeDtypeStruct((B, 68, R), jnp.float32),
    )(g, nx_cm)


def _mlp_body(gt_ref, w0x_ref, w0p_ref, b0_ref, w1_ref, b1_ref,
              wn0_ref, bn0_ref, wn1_ref, bn1_ref, wn2_ref, bn2_ref,
              lin_ref, lb_ref, out_ref):
    R = NSAMPLE * NPOINT

    def inorm_relu(h):
        mu = jnp.mean(h, axis=1, keepdims=True)
        v = jnp.mean((h - mu) * (h - mu), axis=1, keepdims=True)
        return jnp.maximum((h - mu) / jnp.sqrt(v + EPS), 0.0)

    def mm(a, b):
        return jax.lax.dot_general(a, b, (((1,), (0,)), ((), ())),
                                   preferred_element_type=jnp.float32)

    gt = gt_ref[...]
    gxyzn = gt[0:3, :]
    dens = gt[3:4, :]
    feats = gt[4:68, :]

    h = mm(w0x_ref[...], gxyzn) + mm(w0p_ref[...], feats) + b0_ref[...]
    h = inorm_relu(h)
    x1 = inorm_relu(mm(w1_ref[...], h) + b1_ref[...])

    mx = dens[:, 0:NPOINT]
    for s in range(1, NSAMPLE):
        mx = jnp.maximum(mx, dens[:, s * NPOINT:(s + 1) * NPOINT])
    mx = jnp.maximum(mx, 1e-12)
    dsc = jnp.concatenate(
        [dens[:, s * NPOINT:(s + 1) * NPOINT] / mx for s in range(NSAMPLE)],
        axis=1)
    x1 = x1 * dsc

    w = inorm_relu(mm(wn0_ref[...], gxyzn) + bn0_ref[...])
    w = inorm_relu(mm(wn1_ref[...], w) + bn1_ref[...])
    w = inorm_relu(mm(wn2_ref[...], w) + bn2_ref[...])

    lin = lin_ref[...]
    acc = jnp.zeros((128, NPOINT), jnp.float32)
    for f in range(16):
        wf = w[f:f + 1, :]
        s3 = jnp.zeros((128, NPOINT), jnp.float32)
        for s in range(NSAMPLE):
            sl = slice(s * NPOINT, (s + 1) * NPOINT)
            s3 = s3 + x1[:, sl] * wf[:, sl]
        acc = acc + jax.lax.dot_general(
            lin[f], s3, (((1,), (0,)), ((), ())),
            preferred_element_type=jnp.float32)
    acc = acc + lb_ref[...]
    out_ref[...] = inorm_relu(acc)


def _mlp(gt, w0x, w0p, b0, w1, b1, wn0, bn0, wn1, bn1, wn2, bn2,
         lin_stack, lin_b):
    B = gt.shape[0]
    R = NSAMPLE * NPOINT
    wspec = lambda shp: pl.BlockSpec(shp, lambda b: tuple(0 for _ in shp))
    return pl.pallas_call(
        _mlp_body,
        grid=(B,),
        in_specs=[
            pl.BlockSpec((None, 68, R), lambda b: (b, 0, 0)),
            wspec((64, 3)), wspec((64, 64)), wspec((64, 1)),
            wspec((128, 64)), wspec((128, 1)),
            wspec((8, 3)), wspec((8, 1)),
            wspec((8, 8)), wspec((8, 1)),
            wspec((16, 8)), wspec((16, 1)),
            wspec((16, 128, 128)), wspec((128, 1)),
        ],
        out_specs=pl.BlockSpec((None, 128, NPOINT), lambda b: (b, 0, 0)),
        out_shape=jax.ShapeDtypeStruct((B, 128, NPOINT), jnp.float32),
    )(gt, w0x, w0p, b0, w1, b1, wn0, bn0, wn1, bn1, wn2, bn2, lin_stack,
      lin_b)


def kernel(xyz, points, mlp_w0, mlp_b0, mlp_w1, mlp_b1, dn_w0, dn_b0,
           dn_w1, dn_b1, dn_w2, dn_b2, wn_w0, wn_b0, wn_w1, wn_b1, wn_w2,
           wn_b2, lin_w, lin_b):
    B = xyz.shape[0]
    xyz_rows = jnp.transpose(xyz, (0, 2, 1))
    pts_rows = jnp.transpose(points, (0, 2, 1))

    dens = _density(xyz_rows, xyz)
    dscale = _density_scale(dens, dn_w0, dn_b0, dn_w1, dn_b1, dn_w2,
                            dn_b2)

    new_xyz_cm, new_xyz_rows = _fps(xyz)
    knn = _knn(new_xyz_rows, xyz)

    pad = jnp.zeros((B, N, 12), jnp.float32)
    pad2 = jnp.zeros((B, N, 48), jnp.float32)
    table = jnp.concatenate(
        [xyz_rows, dscale[:, :, None], pad, pts_rows, pad2], axis=2
    ).reshape(B * N, 128)

    offs = (jnp.arange(B, dtype=jnp.int32) * N)[:, None, None]
    gidx = (jnp.transpose(knn, (0, 2, 1)) + offs).reshape(-1)
    g = _sc_gather(table, gidx)
    g = g.reshape(B, NSAMPLE * NPOINT, 128)

    lin_stack = jnp.transpose(lin_w.reshape(128, 128, 16), (2, 0, 1))

    gt = _transpose_prep(g, new_xyz_cm)
    out = _mlp(
        gt, mlp_w0[:, 0:3], mlp_w0[:, 3:67], mlp_b0[:, None],
        mlp_w1, mlp_b1[:, None],
        wn_w0, wn_b0[:, None], wn_w1, wn_b1[:, None],
        wn_w2, wn_b2[:, None],
        lin_stack, lin_b[:, None])

    return new_xyz_cm, out
